# Initial kernel scaffold; baseline (speedup 1.0000x reference)
import jax
import jax.numpy as jnp
from jax.experimental import pallas as pl


def kernel(x, edge_index_rel0, edge_index_rel1, W0, b0, W1, b1, prelu_a):
    raise NotImplementedError("write your pallas kernel here")


# speedup vs baseline: 2.8302x; 2.8302x over previous
import functools

import jax
import jax.numpy as jnp
from jax import lax
from jax.experimental import pallas as pl
from jax.experimental.pallas import tpu as pltpu
from jax.experimental.pallas import tpu_sc as plsc

_N = 10000
_NP = 10240
_HN = _NP // 2
_E = 320000
_D = 128
_L = 16
_NS = 16
_RPT = _HN // _NS
_EPT = _E // _NS
_CH = 80
_NCH = _EPT // _CH
_BN = 1000

_mesh = plsc.VectorSubcoreMesh(
    core_axis_name="c", subcore_axis_name="s", num_cores=2, num_subcores=_NS
)


def _deg_body(s0, d0, s1, d1, deg_out, ones_v, idx_v, zero_v, src_tab, dst_tab):
    c = lax.axis_index("c")
    s = lax.axis_index("s")
    one = jnp.full((_L,), 1.0, jnp.float32)
    for k in range(_CH // _L):
        ones_v[pl.ds(k * _L, _L)] = one
    zero = jnp.zeros((_L,), jnp.float32)

    def _z(i, _):
        zero_v[pl.ds(i * _L, _L)] = zero
        return 0

    lax.fori_loop(0, _N // _L, _z, 0)

    @pl.when(s == 0)
    def _():
        pltpu.sync_copy(zero_v, src_tab)

    @pl.when(s == 1)
    def _():
        pltpu.sync_copy(zero_v, dst_tab)

    plsc.subcore_barrier()

    def _hist(es, ed):
        base = s * _EPT

        def _chunk(i, _):
            off = pl.multiple_of(base + i * _CH, 8)
            pltpu.sync_copy(es.at[pl.ds(off, _CH)], idx_v)
            pltpu.sync_copy(ones_v, src_tab.at[idx_v], add=True)
            pltpu.sync_copy(ed.at[pl.ds(off, _CH)], idx_v)
            pltpu.sync_copy(ones_v, dst_tab.at[idx_v], add=True)
            return 0

        lax.fori_loop(0, _NCH, _chunk, 0)

    @pl.when(c == 0)
    def _():
        _hist(s0, d0)

    @pl.when(c == 1)
    def _():
        _hist(s1, d1)

    plsc.subcore_barrier()

    def _writeout(tab, row):
        pltpu.sync_copy(tab, zero_v)
        pltpu.sync_copy(zero_v, deg_out.at[pl.ds(row * _N, _N)])

    @pl.when(c == 0)
    def _():
        @pl.when(s == 0)
        def _():
            _writeout(src_tab, 0)

        @pl.when(s == 1)
        def _():
            _writeout(dst_tab, 1)

    @pl.when(c == 1)
    def _():
        @pl.when(s == 0)
        def _():
            _writeout(src_tab, 2)

        @pl.when(s == 1)
        def _():
            _writeout(dst_tab, 3)


_deg_kernel = functools.partial(
    pl.kernel,
    out_type=jax.ShapeDtypeStruct((4 * _N,), jnp.float32),
    mesh=_mesh,
    scratch_types=[
        pltpu.VMEM((_CH,), jnp.float32),
        pltpu.VMEM((_CH,), jnp.int32),
        pltpu.VMEM((_N,), jnp.float32),
        pltpu.VMEM_SHARED((_N,), jnp.float32),
        pltpu.VMEM_SHARED((_N,), jnp.float32),
    ],
)(_deg_body)


def _edge_body(ys0, ys1, es0, ed0, es1, ed1, agg_out, sidx, didx, fidx, ldidx,
               rows, stg, atab):
    s = lax.axis_index("s")
    c = lax.axis_index("c")
    zero = jnp.zeros((_L,), jnp.float32)
    row0 = s * _RPT

    def _rel(ysr, es, ed, r):
        for p in range(2):
            half0 = p * _HN

            def _zfill(i, _):
                for k in range(_D // _L):
                    stg[i, pl.ds(k * _L, _L)] = zero
                return 0

            lax.fori_loop(0, _RPT, _zfill, 0)
            pltpu.sync_copy(stg, atab.at[pl.ds(row0, _RPT)])
            plsc.subcore_barrier()

            def _chunk(i, _):
                off = pl.multiple_of(s * _EPT + i * _CH, 8)
                pltpu.sync_copy(es.at[pl.ds(off, _CH)], sidx)
                pltpu.sync_copy(ed.at[pl.ds(off, _CH)], didx)
                for k in range(_CH // _L):
                    sl = pl.ds(k * _L, _L)
                    lv = didx[sl] - half0
                    keep = (lv >= 0) & (lv < _HN)
                    ldidx[sl] = jnp.where(keep, lv, -1)
                    fidx[sl] = jnp.where(keep, sidx[sl], -1)
                pltpu.sync_copy(
                    ysr.at[plsc.Indices(fidx, ignored_value=-1)], rows
                )
                pltpu.sync_copy(
                    rows, atab.at[plsc.Indices(ldidx, ignored_value=-1)],
                    add=True,
                )
                return 0

            lax.fori_loop(0, _NCH, _chunk, 0)
            plsc.subcore_barrier()
            pltpu.sync_copy(atab.at[pl.ds(row0, _RPT)], stg)
            pltpu.sync_copy(stg, agg_out.at[r, pl.ds(half0 + row0, _RPT)])

    @pl.when(c == 0)
    def _():
        _rel(ys0, es0, ed0, 0)

    @pl.when(c == 1)
    def _():
        _rel(ys1, es1, ed1, 1)


_edge_kernel = functools.partial(
    pl.kernel,
    out_type=jax.ShapeDtypeStruct((2, _NP, _D), jnp.float32),
    mesh=_mesh,
    scratch_types=[
        pltpu.VMEM((_CH,), jnp.int32),
        pltpu.VMEM((_CH,), jnp.int32),
        pltpu.VMEM((_CH,), jnp.int32),
        pltpu.VMEM((_CH,), jnp.int32),
        pltpu.VMEM((_CH, _D), jnp.float32),
        pltpu.VMEM((_RPT, _D), jnp.float32),
        pltpu.VMEM_SHARED((_HN, _D), jnp.float32),
    ],
)(_edge_body)


def _scale_body(x_ref, w_ref, deg_ref, y0_ref, y1_ref):
    y = jax.lax.dot_general(
        x_ref[...], w_ref[...], (((1,), (0,)), ((), ())),
        preferred_element_type=jnp.float32,
        precision=jax.lax.Precision.HIGHEST,
    )
    d0 = deg_ref[:, 0:1]
    ns0 = lax.rsqrt(jnp.where(d0 > 0, d0, 1.0))
    d1 = deg_ref[:, 2:3]
    ns1 = lax.rsqrt(jnp.where(d1 > 0, d1, 1.0))
    y0_ref[...] = y[:, :_D] * ns0
    y1_ref[...] = y[:, _D:] * ns1


_scale_kernel = pl.pallas_call(
    _scale_body,
    grid=(_N // _BN,),
    in_specs=[
        pl.BlockSpec((_BN, _D), lambda i: (i, 0)),
        pl.BlockSpec((_D, 2 * _D), lambda i: (0, 0)),
        pl.BlockSpec((_BN, 4), lambda i: (i, 0)),
    ],
    out_specs=[
        pl.BlockSpec((_BN, _D), lambda i: (i, 0)),
        pl.BlockSpec((_BN, _D), lambda i: (i, 0)),
    ],
    out_shape=[
        jax.ShapeDtypeStruct((_NP, _D), jnp.float32),
        jax.ShapeDtypeStruct((_NP, _D), jnp.float32),
    ],
)


def _final_body(agg_ref, deg_ref, b0_ref, b1_ref, a_ref, out_ref):
    d0 = deg_ref[:, 1:2]
    nd0 = lax.rsqrt(jnp.where(d0 > 0, d0, 1.0))
    d1 = deg_ref[:, 3:4]
    nd1 = lax.rsqrt(jnp.where(d1 > 0, d1, 1.0))
    h = agg_ref[0] * nd0 + agg_ref[1] * nd1 + b0_ref[...] + b1_ref[...]
    out_ref[...] = jnp.where(h > 0, h, a_ref[...] * h)


_final_kernel = pl.pallas_call(
    _final_body,
    grid=(_N // _BN,),
    in_specs=[
        pl.BlockSpec((2, _BN, _D), lambda i: (0, i, 0)),
        pl.BlockSpec((_BN, 4), lambda i: (i, 0)),
        pl.BlockSpec((1, _D), lambda i: (0, 0)),
        pl.BlockSpec((1, _D), lambda i: (0, 0)),
        pl.BlockSpec((1, 1), lambda i: (0, 0)),
    ],
    out_specs=pl.BlockSpec((_BN, _D), lambda i: (i, 0)),
    out_shape=jax.ShapeDtypeStruct((_N, _D), jnp.float32),
)


def kernel(x, edge_index_rel0, edge_index_rel1, W0, b0, W1, b1, prelu_a):
    es0, ed0 = edge_index_rel0[0], edge_index_rel0[1]
    es1, ed1 = edge_index_rel1[0], edge_index_rel1[1]
    deg = _deg_kernel(es0, ed0, es1, ed1).reshape(4, _N)
    deg_t = deg.T
    w_cat = jnp.concatenate([W0, W1], axis=1)
    ys0, ys1 = _scale_kernel(x, w_cat, deg_t)
    agg = _edge_kernel(ys0, ys1, es0, ed0, es1, ed1)
    return _final_kernel(
        agg, deg_t, b0.reshape(1, _D), b1.reshape(1, _D), prelu_a.reshape(1, 1)
    )


# speedup vs baseline: 11.3696x; 4.0172x over previous
import functools

import jax
import jax.numpy as jnp
from jax import lax
from jax.experimental import pallas as pl
from jax.experimental.pallas import tpu as pltpu
from jax.experimental.pallas import tpu_sc as plsc

_N = 10000
_NP = 10240
_HN = _NP // 2
_E = 320000
_D = 128
_L = 16
_NS = 16
_RPT = _HN // _NS
_EPT = _E // _NS
_CH = 80
_NCH = _EPT // _CH
_BN = 1000

_mesh = plsc.VectorSubcoreMesh(
    core_axis_name="c", subcore_axis_name="s", num_cores=2, num_subcores=_NS
)


def _deg_body(s0, d0, s1, d1, deg_out, ones_v, zero_v, si0, si1, si2, si3,
              di0, di1, di2, di3, semi0, semi1, semi2, semi3,
              sems0, sems1, sems2, sems3, semd0, semd1, semd2, semd3,
              src_tab, dst_tab):
    si = (si0, si1, si2, si3)
    di = (di0, di1, di2, di3)
    semi = (semi0, semi1, semi2, semi3)
    sems = (sems0, sems1, sems2, sems3)
    semd = (semd0, semd1, semd2, semd3)
    c = lax.axis_index("c")
    s = lax.axis_index("s")
    one = jnp.full((_L,), 1.0, jnp.float32)
    for k in range(_CH // _L):
        ones_v[pl.ds(k * _L, _L)] = one
    zero = jnp.zeros((_L,), jnp.float32)

    def _z(i, _):
        zero_v[pl.ds(i * _L, _L)] = zero
        return 0

    lax.fori_loop(0, _N // _L, _z, 0)

    @pl.when(s == 0)
    def _():
        pltpu.sync_copy(zero_v, src_tab)

    @pl.when(s == 1)
    def _():
        pltpu.sync_copy(zero_v, dst_tab)

    plsc.subcore_barrier()

    def _hist(es, ed, si, di, semi, sems, semd):
        base = s * _EPT

        def _ldidx(i, b):
            off = pl.multiple_of(base + i * _CH, 8)
            pltpu.async_copy(es.at[pl.ds(off, _CH)], si[b], semi[b])
            pltpu.async_copy(ed.at[pl.ds(off, _CH)], di[b], semi[b])

        for b in range(4):
            _ldidx(b, b)

        def _loop(j, _):
            for b in range(4):
                i = j * 4 + b
                pltpu.make_async_copy(es.at[pl.ds(0, _CH)], si[b], semi[b]).wait()
                pltpu.make_async_copy(es.at[pl.ds(0, _CH)], di[b], semi[b]).wait()

                @pl.when(j >= 1)
                def _():
                    pltpu.make_async_copy(ones_v, src_tab.at[si[b]], sems[b]).wait()
                    pltpu.make_async_copy(ones_v, dst_tab.at[di[b]], semd[b]).wait()

                pltpu.async_copy(ones_v, src_tab.at[si[b]], sems[b], add=True)
                pltpu.async_copy(ones_v, dst_tab.at[di[b]], semd[b], add=True)

                @pl.when(j < _NCH // 4 - 1)
                def _():
                    _ldidx(i + 4, b)

            return 0

        lax.fori_loop(0, _NCH // 4, _loop, 0)
        for b in range(4):
            pltpu.make_async_copy(ones_v, src_tab.at[si[b]], sems[b]).wait()
            pltpu.make_async_copy(ones_v, dst_tab.at[di[b]], semd[b]).wait()
        for i in range(4 * (_NCH // 4), _NCH):
            off = pl.multiple_of(base + i * _CH, 8)
            pltpu.sync_copy(es.at[pl.ds(off, _CH)], si[0])
            pltpu.sync_copy(ones_v, src_tab.at[si[0]], add=True)
            pltpu.sync_copy(ed.at[pl.ds(off, _CH)], di[0])
            pltpu.sync_copy(ones_v, dst_tab.at[di[0]], add=True)

    @pl.when(c == 0)
    def _():
        _hist(s0, d0, si, di, semi, sems, semd)

    @pl.when(c == 1)
    def _():
        _hist(s1, d1, si, di, semi, sems, semd)

    plsc.subcore_barrier()

    def _writeout(tab, row):
        pltpu.sync_copy(tab, zero_v)
        pltpu.sync_copy(zero_v, deg_out.at[pl.ds(row * _N, _N)])

    @pl.when(c == 0)
    def _():
        @pl.when(s == 0)
        def _():
            _writeout(src_tab, 0)

        @pl.when(s == 1)
        def _():
            _writeout(dst_tab, 1)

    @pl.when(c == 1)
    def _():
        @pl.when(s == 0)
        def _():
            _writeout(src_tab, 2)

        @pl.when(s == 1)
        def _():
            _writeout(dst_tab, 3)


_deg_kernel = functools.partial(
    pl.kernel,
    out_type=jax.ShapeDtypeStruct((4 * _N,), jnp.float32),
    mesh=_mesh,
    scratch_types=[
        pltpu.VMEM((_CH,), jnp.float32),
        pltpu.VMEM((_N,), jnp.float32),
    ]
    + [pltpu.VMEM((_CH,), jnp.int32) for _ in range(8)]
    + [pltpu.SemaphoreType.DMA for _ in range(12)]
    + [
        pltpu.VMEM_SHARED((_N,), jnp.float32),
        pltpu.VMEM_SHARED((_N,), jnp.float32),
    ],
)(_deg_body)


def _edge_body(ys0, ys1, es0, ed0, es1, ed1, agg_out, stg,
               sx0, sx1, sx2, sx3, dx0, dx1, dx2, dx3,
               fx0, fx1, fx2, fx3, lx0, lx1, lx2, lx3,
               rw0, rw1, rw2, rw3,
               mi0, mi1, mi2, mi3, mg0, mg1, mg2, mg3, ms0, ms1, ms2, ms3,
               atab):
    sidx = (sx0, sx1, sx2, sx3)
    didx = (dx0, dx1, dx2, dx3)
    fidx = (fx0, fx1, fx2, fx3)
    ldidx = (lx0, lx1, lx2, lx3)
    rows = (rw0, rw1, rw2, rw3)
    semi = (mi0, mi1, mi2, mi3)
    semg = (mg0, mg1, mg2, mg3)
    sems = (ms0, ms1, ms2, ms3)
    s = lax.axis_index("s")
    c = lax.axis_index("c")
    zero = jnp.zeros((_L,), jnp.float32)
    row0 = s * _RPT
    base = s * _EPT

    def _rel(ysr, es, ed, r):
        def _ldidx(i, b):
            off = pl.multiple_of(base + i * _CH, 8)
            pltpu.async_copy(es.at[pl.ds(off, _CH)], sidx[b], semi[b])
            pltpu.async_copy(ed.at[pl.ds(off, _CH)], didx[b], semi[b])

        def _filter(b, half0):
            for k in range(_CH // _L):
                sl = pl.ds(k * _L, _L)
                lv = didx[b][sl] - half0
                keep = (lv >= 0) & (lv < _HN)
                ldidx[b][sl] = jnp.where(keep, lv, -1)
                fidx[b][sl] = jnp.where(keep, sidx[b][sl], -1)

        def _gather(ysr, b):
            pltpu.async_copy(
                ysr.at[plsc.Indices(fidx[b], ignored_value=-1)], rows[b],
                semg[b],
            )

        def _scat_desc(b):
            return pltpu.make_async_copy(
                rows[b], atab.at[plsc.Indices(ldidx[b], ignored_value=-1)],
                sems[b],
            )

        for p in range(2):
            half0 = p * _HN

            def _zfill(i, _):
                for k in range(_D // _L):
                    stg[i, pl.ds(k * _L, _L)] = zero
                return 0

            lax.fori_loop(0, _RPT, _zfill, 0)
            pltpu.sync_copy(stg, atab.at[pl.ds(row0, _RPT)])
            plsc.subcore_barrier()

            for b in range(4):
                _ldidx(b, b)

            def _loop(j, _):
                for b in range(4):
                    i = j * 4 + b
                    pltpu.make_async_copy(
                        es.at[pl.ds(0, _CH)], sidx[b], semi[b]).wait()
                    pltpu.make_async_copy(
                        es.at[pl.ds(0, _CH)], didx[b], semi[b]).wait()

                    @pl.when(j >= 1)
                    def _():
                        _scat_desc(b).wait()

                    _filter(b, half0)
                    _gather(ysr, b)

                    @pl.when(j < _NCH // 4 - 1)
                    def _():
                        _ldidx(i + 4, b)

                for b in range(4):
                    pltpu.make_async_copy(
                        ysr.at[plsc.Indices(fidx[b], ignored_value=-1)],
                        rows[b], semg[b]).wait()
                    pltpu.async_copy(
                        rows[b],
                        atab.at[plsc.Indices(ldidx[b], ignored_value=-1)],
                        sems[b], add=True,
                    )

                return 0

            lax.fori_loop(0, _NCH // 4, _loop, 0)
            for b in range(4):
                _scat_desc(b).wait()
            for i in range(4 * (_NCH // 4), _NCH):
                off = pl.multiple_of(base + i * _CH, 8)
                pltpu.sync_copy(es.at[pl.ds(off, _CH)], sidx[0])
                pltpu.sync_copy(ed.at[pl.ds(off, _CH)], didx[0])
                _filter(0, half0)
                pltpu.sync_copy(
                    ysr.at[plsc.Indices(fidx[0], ignored_value=-1)], rows[0]
                )
                pltpu.sync_copy(
                    rows[0], atab.at[plsc.Indices(ldidx[0], ignored_value=-1)],
                    add=True,
                )
            plsc.subcore_barrier()
            pltpu.sync_copy(atab.at[pl.ds(row0, _RPT)], stg)
            pltpu.sync_copy(stg, agg_out.at[r, pl.ds(half0 + row0, _RPT)])

    @pl.when(c == 0)
    def _():
        _rel(ys0, es0, ed0, 0)

    @pl.when(c == 1)
    def _():
        _rel(ys1, es1, ed1, 1)


_edge_kernel = functools.partial(
    pl.kernel,
    out_type=jax.ShapeDtypeStruct((2, _NP, _D), jnp.float32),
    mesh=_mesh,
    scratch_types=[pltpu.VMEM((_RPT, _D), jnp.float32)]
    + [pltpu.VMEM((_CH,), jnp.int32) for _ in range(16)]
    + [pltpu.VMEM((_CH, _D), jnp.float32) for _ in range(4)]
    + [pltpu.SemaphoreType.DMA for _ in range(12)]
    + [pltpu.VMEM_SHARED((_HN, _D), jnp.float32)],
)(_edge_body)


def _scale_body(x_ref, w_ref, deg_ref, y0_ref, y1_ref):
    y = jax.lax.dot_general(
        x_ref[...], w_ref[...], (((1,), (0,)), ((), ())),
        preferred_element_type=jnp.float32,
        precision=jax.lax.Precision.HIGHEST,
    )
    d0 = deg_ref[:, 0:1]
    ns0 = lax.rsqrt(jnp.where(d0 > 0, d0, 1.0))
    d1 = deg_ref[:, 2:3]
    ns1 = lax.rsqrt(jnp.where(d1 > 0, d1, 1.0))
    y0_ref[...] = y[:, :_D] * ns0
    y1_ref[...] = y[:, _D:] * ns1


_scale_kernel = pl.pallas_call(
    _scale_body,
    grid=(_N // _BN,),
    in_specs=[
        pl.BlockSpec((_BN, _D), lambda i: (i, 0)),
        pl.BlockSpec((_D, 2 * _D), lambda i: (0, 0)),
        pl.BlockSpec((_BN, 4), lambda i: (i, 0)),
    ],
    out_specs=[
        pl.BlockSpec((_BN, _D), lambda i: (i, 0)),
        pl.BlockSpec((_BN, _D), lambda i: (i, 0)),
    ],
    out_shape=[
        jax.ShapeDtypeStruct((_NP, _D), jnp.float32),
        jax.ShapeDtypeStruct((_NP, _D), jnp.float32),
    ],
)


def _final_body(agg_ref, deg_ref, b0_ref, b1_ref, a_ref, out_ref):
    d0 = deg_ref[:, 1:2]
    nd0 = lax.rsqrt(jnp.where(d0 > 0, d0, 1.0))
    d1 = deg_ref[:, 3:4]
    nd1 = lax.rsqrt(jnp.where(d1 > 0, d1, 1.0))
    h = agg_ref[0] * nd0 + agg_ref[1] * nd1 + b0_ref[...] + b1_ref[...]
    out_ref[...] = jnp.where(h > 0, h, a_ref[...] * h)


_final_kernel = pl.pallas_call(
    _final_body,
    grid=(_N // _BN,),
    in_specs=[
        pl.BlockSpec((2, _BN, _D), lambda i: (0, i, 0)),
        pl.BlockSpec((_BN, 4), lambda i: (i, 0)),
        pl.BlockSpec((1, _D), lambda i: (0, 0)),
        pl.BlockSpec((1, _D), lambda i: (0, 0)),
        pl.BlockSpec((1, 1), lambda i: (0, 0)),
    ],
    out_specs=pl.BlockSpec((_BN, _D), lambda i: (i, 0)),
    out_shape=jax.ShapeDtypeStruct((_N, _D), jnp.float32),
)


def kernel(x, edge_index_rel0, edge_index_rel1, W0, b0, W1, b1, prelu_a):
    es0, ed0 = edge_index_rel0[0], edge_index_rel0[1]
    es1, ed1 = edge_index_rel1[0], edge_index_rel1[1]
    deg = _deg_kernel(es0, ed0, es1, ed1).reshape(4, _N)
    deg_t = deg.T
    w_cat = jnp.concatenate([W0, W1], axis=1)
    ys0, ys1 = _scale_kernel(x, w_cat, deg_t)
    agg = _edge_kernel(ys0, ys1, es0, ed0, es1, ed1)
    return _final_kernel(
        agg, deg_t, b0.reshape(1, _D), b1.reshape(1, _D), prelu_a.reshape(1, 1)
    )


# speedup vs baseline: 12.5100x; 1.1003x over previous
import functools

import jax
import jax.numpy as jnp
from jax import lax
from jax.experimental import pallas as pl
from jax.experimental.pallas import tpu as pltpu
from jax.experimental.pallas import tpu_sc as plsc

_N = 10000
_NP = 10240
_HN = _NP // 2
_E = 320000
_D = 128
_L = 16
_NS = 16
_RPT = _HN // _NS
_EPT = _E // _NS
_CH = 80
_NCH = _EPT // _CH
_CHE = 128
_NCHE = _EPT // _CHE
_BN = 1000

_mesh = plsc.VectorSubcoreMesh(
    core_axis_name="c", subcore_axis_name="s", num_cores=2, num_subcores=_NS
)


def _deg_body(s0, d0, s1, d1, deg_out, ones_v, zero_v, si0, si1, si2, si3,
              di0, di1, di2, di3, semi0, semi1, semi2, semi3,
              sems0, sems1, sems2, sems3, semd0, semd1, semd2, semd3,
              src_tab, dst_tab):
    si = (si0, si1, si2, si3)
    di = (di0, di1, di2, di3)
    semi = (semi0, semi1, semi2, semi3)
    sems = (sems0, sems1, sems2, sems3)
    semd = (semd0, semd1, semd2, semd3)
    c = lax.axis_index("c")
    s = lax.axis_index("s")
    one = jnp.full((_L,), 1.0, jnp.float32)
    for k in range(_CH // _L):
        ones_v[pl.ds(k * _L, _L)] = one
    zero = jnp.zeros((_L,), jnp.float32)

    def _z(i, _):
        zero_v[pl.ds(i * _L, _L)] = zero
        return 0

    lax.fori_loop(0, _N // _L, _z, 0)

    @pl.when(s == 0)
    def _():
        pltpu.sync_copy(zero_v, src_tab)

    @pl.when(s == 1)
    def _():
        pltpu.sync_copy(zero_v, dst_tab)

    plsc.subcore_barrier()

    def _hist(es, ed, si, di, semi, sems, semd):
        base = s * _EPT

        def _ldidx(i, b):
            off = pl.multiple_of(base + i * _CH, 8)
            pltpu.async_copy(es.at[pl.ds(off, _CH)], si[b], semi[b])
            pltpu.async_copy(ed.at[pl.ds(off, _CH)], di[b], semi[b])

        for b in range(4):
            _ldidx(b, b)

        def _loop(j, _):
            for b in range(4):
                i = j * 4 + b
                pltpu.make_async_copy(es.at[pl.ds(0, _CH)], si[b], semi[b]).wait()
                pltpu.make_async_copy(es.at[pl.ds(0, _CH)], di[b], semi[b]).wait()

                @pl.when(j >= 1)
                def _():
                    pltpu.make_async_copy(ones_v, src_tab.at[si[b]], sems[b]).wait()
                    pltpu.make_async_copy(ones_v, dst_tab.at[di[b]], semd[b]).wait()

                pltpu.async_copy(ones_v, src_tab.at[si[b]], sems[b], add=True)
                pltpu.async_copy(ones_v, dst_tab.at[di[b]], semd[b], add=True)

                @pl.when(j < _NCH // 4 - 1)
                def _():
                    _ldidx(i + 4, b)

            return 0

        lax.fori_loop(0, _NCH // 4, _loop, 0)
        for b in range(4):
            pltpu.make_async_copy(ones_v, src_tab.at[si[b]], sems[b]).wait()
            pltpu.make_async_copy(ones_v, dst_tab.at[di[b]], semd[b]).wait()
        for i in range(4 * (_NCH // 4), _NCH):
            off = pl.multiple_of(base + i * _CH, 8)
            pltpu.sync_copy(es.at[pl.ds(off, _CH)], si[0])
            pltpu.sync_copy(ones_v, src_tab.at[si[0]], add=True)
            pltpu.sync_copy(ed.at[pl.ds(off, _CH)], di[0])
            pltpu.sync_copy(ones_v, dst_tab.at[di[0]], add=True)

    @pl.when(c == 0)
    def _():
        _hist(s0, d0, si, di, semi, sems, semd)

    @pl.when(c == 1)
    def _():
        _hist(s1, d1, si, di, semi, sems, semd)

    plsc.subcore_barrier()

    def _writeout(tab, row):
        pltpu.sync_copy(tab, zero_v)
        pltpu.sync_copy(zero_v, deg_out.at[pl.ds(row * _N, _N)])

    @pl.when(c == 0)
    def _():
        @pl.when(s == 0)
        def _():
            _writeout(src_tab, 0)

        @pl.when(s == 1)
        def _():
            _writeout(dst_tab, 1)

    @pl.when(c == 1)
    def _():
        @pl.when(s == 0)
        def _():
            _writeout(src_tab, 2)

        @pl.when(s == 1)
        def _():
            _writeout(dst_tab, 3)


_deg_kernel = functools.partial(
    pl.kernel,
    out_type=jax.ShapeDtypeStruct((4 * _N,), jnp.float32),
    mesh=_mesh,
    scratch_types=[
        pltpu.VMEM((_CH,), jnp.float32),
        pltpu.VMEM((_N,), jnp.float32),
    ]
    + [pltpu.VMEM((_CH,), jnp.int32) for _ in range(8)]
    + [pltpu.SemaphoreType.DMA for _ in range(12)]
    + [
        pltpu.VMEM_SHARED((_N,), jnp.float32),
        pltpu.VMEM_SHARED((_N,), jnp.float32),
    ],
)(_deg_body)


def _edge_body(ys0, ys1, es0, ed0, es1, ed1, agg_out, stg,
               sx0, sx1, sx2, sx3, dx0, dx1, dx2, dx3,
               fx0, fx1, fx2, fx3, lx0, lx1, lx2, lx3,
               rw0, rw1, rw2, rw3,
               mi0, mi1, mi2, mi3, mg0, mg1, mg2, mg3, ms0, ms1, ms2, ms3,
               atab):
    sidx = (sx0, sx1, sx2, sx3)
    didx = (dx0, dx1, dx2, dx3)
    fidx = (fx0, fx1, fx2, fx3)
    ldidx = (lx0, lx1, lx2, lx3)
    rows = (rw0, rw1, rw2, rw3)
    semi = (mi0, mi1, mi2, mi3)
    semg = (mg0, mg1, mg2, mg3)
    sems = (ms0, ms1, ms2, ms3)
    s = lax.axis_index("s")
    c = lax.axis_index("c")
    zero = jnp.zeros((_L,), jnp.float32)
    row0 = s * _RPT
    base = s * _EPT

    def _rel(ysr, es, ed, r):
        def _ldidx(i, b):
            off = pl.multiple_of(base + i * _CHE, 8)
            pltpu.async_copy(es.at[pl.ds(off, _CHE)], sidx[b], semi[b])
            pltpu.async_copy(ed.at[pl.ds(off, _CHE)], didx[b], semi[b])

        def _filter(b, half0):
            for k in range(_CHE // _L):
                sl = pl.ds(k * _L, _L)
                lv = didx[b][sl] - half0
                keep = (lv >= 0) & (lv < _HN)
                ldidx[b][sl] = jnp.where(keep, lv, -1)
                fidx[b][sl] = jnp.where(keep, sidx[b][sl], -1)

        def _gather(ysr, b):
            pltpu.async_copy(
                ysr.at[plsc.Indices(fidx[b], ignored_value=-1)], rows[b],
                semg[b],
            )

        def _scat_desc(b):
            return pltpu.make_async_copy(
                rows[b], atab.at[plsc.Indices(ldidx[b], ignored_value=-1)],
                sems[b],
            )

        for p in range(2):
            half0 = p * _HN

            def _zfill(i, _):
                for k in range(_D // _L):
                    stg[i, pl.ds(k * _L, _L)] = zero
                return 0

            lax.fori_loop(0, _RPT // 2, _zfill, 0)
            for q in range(2):
                pltpu.sync_copy(
                    stg, atab.at[pl.ds(row0 + q * (_RPT // 2), _RPT // 2)])
            plsc.subcore_barrier()

            for b in range(4):
                _ldidx(b, b)

            def _loop(j, _):
                for b in range(4):
                    i = j * 4 + b
                    pltpu.make_async_copy(
                        es.at[pl.ds(0, _CHE)], sidx[b], semi[b]).wait()
                    pltpu.make_async_copy(
                        es.at[pl.ds(0, _CHE)], didx[b], semi[b]).wait()

                    @pl.when(j >= 1)
                    def _():
                        _scat_desc(b).wait()

                    _filter(b, half0)
                    _gather(ysr, b)

                    @pl.when(j < _NCHE // 4 - 1)
                    def _():
                        _ldidx(i + 4, b)

                for b in range(4):
                    pltpu.make_async_copy(
                        ysr.at[plsc.Indices(fidx[b], ignored_value=-1)],
                        rows[b], semg[b]).wait()
                    pltpu.async_copy(
                        rows[b],
                        atab.at[plsc.Indices(ldidx[b], ignored_value=-1)],
                        sems[b], add=True,
                    )

                return 0

            lax.fori_loop(0, _NCHE // 4, _loop, 0)
            for b in range(4):
                _scat_desc(b).wait()
            toff = pl.multiple_of(base + _NCHE * _CHE, 8)
            pltpu.sync_copy(es.at[pl.ds(toff, 32)], sidx[0].at[pl.ds(0, 32)])
            pltpu.sync_copy(ed.at[pl.ds(toff, 32)], didx[0].at[pl.ds(0, 32)])
            sent = jnp.full((_L,), -_NP, jnp.int32)
            for k in range(2, _CHE // _L):
                didx[0][pl.ds(k * _L, _L)] = sent
            _filter(0, half0)
            pltpu.sync_copy(
                ysr.at[plsc.Indices(fidx[0], ignored_value=-1)], rows[0]
            )
            pltpu.sync_copy(
                rows[0], atab.at[plsc.Indices(ldidx[0], ignored_value=-1)],
                add=True,
            )
            plsc.subcore_barrier()
            for q in range(2):
                pltpu.sync_copy(
                    atab.at[pl.ds(row0 + q * (_RPT // 2), _RPT // 2)], stg)
                pltpu.sync_copy(
                    stg,
                    agg_out.at[r, pl.ds(half0 + row0 + q * (_RPT // 2),
                                        _RPT // 2)])

    @pl.when(c == 0)
    def _():
        _rel(ys0, es0, ed0, 0)

    @pl.when(c == 1)
    def _():
        _rel(ys1, es1, ed1, 1)


_edge_kernel = functools.partial(
    pl.kernel,
    out_type=jax.ShapeDtypeStruct((2, _NP, _D), jnp.float32),
    mesh=_mesh,
    scratch_types=[pltpu.VMEM((_RPT // 2, _D), jnp.float32)]
    + [pltpu.VMEM((_CHE,), jnp.int32) for _ in range(16)]
    + [pltpu.VMEM((_CHE, _D), jnp.float32) for _ in range(4)]
    + [pltpu.SemaphoreType.DMA for _ in range(12)]
    + [pltpu.VMEM_SHARED((_HN, _D), jnp.float32)],
)(_edge_body)


def _scale_body(x_ref, w_ref, deg_ref, y0_ref, y1_ref):
    y = jax.lax.dot_general(
        x_ref[...], w_ref[...], (((1,), (0,)), ((), ())),
        preferred_element_type=jnp.float32,
        precision=jax.lax.Precision.HIGHEST,
    )
    d0 = deg_ref[:, 0:1]
    ns0 = lax.rsqrt(jnp.where(d0 > 0, d0, 1.0))
    d1 = deg_ref[:, 2:3]
    ns1 = lax.rsqrt(jnp.where(d1 > 0, d1, 1.0))
    y0_ref[...] = y[:, :_D] * ns0
    y1_ref[...] = y[:, _D:] * ns1


_scale_kernel = pl.pallas_call(
    _scale_body,
    grid=(_N // _BN,),
    in_specs=[
        pl.BlockSpec((_BN, _D), lambda i: (i, 0)),
        pl.BlockSpec((_D, 2 * _D), lambda i: (0, 0)),
        pl.BlockSpec((_BN, 4), lambda i: (i, 0)),
    ],
    out_specs=[
        pl.BlockSpec((_BN, _D), lambda i: (i, 0)),
        pl.BlockSpec((_BN, _D), lambda i: (i, 0)),
    ],
    out_shape=[
        jax.ShapeDtypeStruct((_NP, _D), jnp.float32),
        jax.ShapeDtypeStruct((_NP, _D), jnp.float32),
    ],
)


def _final_body(agg_ref, deg_ref, b0_ref, b1_ref, a_ref, out_ref):
    d0 = deg_ref[:, 1:2]
    nd0 = lax.rsqrt(jnp.where(d0 > 0, d0, 1.0))
    d1 = deg_ref[:, 3:4]
    nd1 = lax.rsqrt(jnp.where(d1 > 0, d1, 1.0))
    h = agg_ref[0] * nd0 + agg_ref[1] * nd1 + b0_ref[...] + b1_ref[...]
    out_ref[...] = jnp.where(h > 0, h, a_ref[...] * h)


_final_kernel = pl.pallas_call(
    _final_body,
    grid=(_N // _BN,),
    in_specs=[
        pl.BlockSpec((2, _BN, _D), lambda i: (0, i, 0)),
        pl.BlockSpec((_BN, 4), lambda i: (i, 0)),
        pl.BlockSpec((1, _D), lambda i: (0, 0)),
        pl.BlockSpec((1, _D), lambda i: (0, 0)),
        pl.BlockSpec((1, 1), lambda i: (0, 0)),
    ],
    out_specs=pl.BlockSpec((_BN, _D), lambda i: (i, 0)),
    out_shape=jax.ShapeDtypeStruct((_N, _D), jnp.float32),
)


def kernel(x, edge_index_rel0, edge_index_rel1, W0, b0, W1, b1, prelu_a):
    es0, ed0 = edge_index_rel0[0], edge_index_rel0[1]
    es1, ed1 = edge_index_rel1[0], edge_index_rel1[1]
    deg = _deg_kernel(es0, ed0, es1, ed1).reshape(4, _N)
    deg_t = deg.T
    w_cat = jnp.concatenate([W0, W1], axis=1)
    ys0, ys1 = _scale_kernel(x, w_cat, deg_t)
    agg = _edge_kernel(ys0, ys1, es0, ed0, es1, ed1)
    return _final_kernel(
        agg, deg_t, b0.reshape(1, _D), b1.reshape(1, _D), prelu_a.reshape(1, 1)
    )


# speedup vs baseline: 12.5184x; 1.0007x over previous
import functools

import jax
import jax.numpy as jnp
from jax import lax
from jax.experimental import pallas as pl
from jax.experimental.pallas import tpu as pltpu
from jax.experimental.pallas import tpu_sc as plsc

_N = 10000
_NP = 10240
_HN = _NP // 2
_E = 320000
_D = 128
_L = 16
_NS = 16
_RPT = _HN // _NS
_EPT = _E // _NS
_CH = 80
_NCH = _EPT // _CH
_CHE = 128
_NCHE = _EPT // _CHE
_BN = 1000

_mesh = plsc.VectorSubcoreMesh(
    core_axis_name="c", subcore_axis_name="s", num_cores=2, num_subcores=_NS
)


def _deg_body(s0, d0, s1, d1, deg_out, ones_v, zero_v, si0, si1, si2, si3,
              di0, di1, di2, di3, semi0, semi1, semi2, semi3,
              sems0, sems1, sems2, sems3, semd0, semd1, semd2, semd3,
              src_tab, dst_tab):
    si = (si0, si1, si2, si3)
    di = (di0, di1, di2, di3)
    semi = (semi0, semi1, semi2, semi3)
    sems = (sems0, sems1, sems2, sems3)
    semd = (semd0, semd1, semd2, semd3)
    c = lax.axis_index("c")
    s = lax.axis_index("s")
    one = jnp.full((_L,), 1.0, jnp.float32)
    for k in range(_CH // _L):
        ones_v[pl.ds(k * _L, _L)] = one
    zero = jnp.zeros((_L,), jnp.float32)

    def _z(i, _):
        zero_v[pl.ds(i * _L, _L)] = zero
        return 0

    lax.fori_loop(0, _N // _L, _z, 0)

    @pl.when(s == 0)
    def _():
        pltpu.sync_copy(zero_v, src_tab)

    @pl.when(s == 1)
    def _():
        pltpu.sync_copy(zero_v, dst_tab)

    plsc.subcore_barrier()

    def _hist(es, ed, si, di, semi, sems, semd):
        base = s * _EPT

        def _ldidx(i, b):
            off = pl.multiple_of(base + i * _CH, 8)
            pltpu.async_copy(es.at[pl.ds(off, _CH)], si[b], semi[b])
            pltpu.async_copy(ed.at[pl.ds(off, _CH)], di[b], semi[b])

        for b in range(4):
            _ldidx(b, b)

        def _loop(j, _):
            for b in range(4):
                i = j * 4 + b
                pltpu.make_async_copy(es.at[pl.ds(0, _CH)], si[b], semi[b]).wait()
                pltpu.make_async_copy(es.at[pl.ds(0, _CH)], di[b], semi[b]).wait()

                @pl.when(j >= 1)
                def _():
                    pltpu.make_async_copy(ones_v, src_tab.at[si[b]], sems[b]).wait()
                    pltpu.make_async_copy(ones_v, dst_tab.at[di[b]], semd[b]).wait()

                pltpu.async_copy(ones_v, src_tab.at[si[b]], sems[b], add=True)
                pltpu.async_copy(ones_v, dst_tab.at[di[b]], semd[b], add=True)

                @pl.when(j < _NCH // 4 - 1)
                def _():
                    _ldidx(i + 4, b)

            return 0

        lax.fori_loop(0, _NCH // 4, _loop, 0)
        for b in range(4):
            pltpu.make_async_copy(ones_v, src_tab.at[si[b]], sems[b]).wait()
            pltpu.make_async_copy(ones_v, dst_tab.at[di[b]], semd[b]).wait()
        for i in range(4 * (_NCH // 4), _NCH):
            off = pl.multiple_of(base + i * _CH, 8)
            pltpu.sync_copy(es.at[pl.ds(off, _CH)], si[0])
            pltpu.sync_copy(ones_v, src_tab.at[si[0]], add=True)
            pltpu.sync_copy(ed.at[pl.ds(off, _CH)], di[0])
            pltpu.sync_copy(ones_v, dst_tab.at[di[0]], add=True)

    @pl.when(c == 0)
    def _():
        _hist(s0, d0, si, di, semi, sems, semd)

    @pl.when(c == 1)
    def _():
        _hist(s1, d1, si, di, semi, sems, semd)

    plsc.subcore_barrier()

    def _writeout(tab, row):
        pltpu.sync_copy(tab, zero_v)
        pltpu.sync_copy(zero_v, deg_out.at[pl.ds(row * _N, _N)])

    @pl.when(c == 0)
    def _():
        @pl.when(s == 0)
        def _():
            _writeout(src_tab, 0)

        @pl.when(s == 1)
        def _():
            _writeout(dst_tab, 1)

    @pl.when(c == 1)
    def _():
        @pl.when(s == 0)
        def _():
            _writeout(src_tab, 2)

        @pl.when(s == 1)
        def _():
            _writeout(dst_tab, 3)


_deg_kernel = functools.partial(
    pl.kernel,
    out_type=jax.ShapeDtypeStruct((4 * _N,), jnp.float32),
    mesh=_mesh,
    scratch_types=[
        pltpu.VMEM((_CH,), jnp.float32),
        pltpu.VMEM((_N,), jnp.float32),
    ]
    + [pltpu.VMEM((_CH,), jnp.int32) for _ in range(8)]
    + [pltpu.SemaphoreType.DMA for _ in range(12)]
    + [
        pltpu.VMEM_SHARED((_N,), jnp.float32),
        pltpu.VMEM_SHARED((_N,), jnp.float32),
    ],
)(_deg_body)


def _edge_body(ys0, ys1, es0, ed0, es1, ed1, agg_out, stg,
               sx0, sx1, sx2, sx3, dx0, dx1, dx2, dx3,
               fx0, fx1, fx2, fx3, lx0, lx1, lx2, lx3,
               rw0, rw1, rw2, rw3,
               mi0, mi1, mi2, mi3, mg0, mg1, mg2, mg3, ms0, ms1, ms2, ms3,
               atab):
    sidx = (sx0, sx1, sx2, sx3)
    didx = (dx0, dx1, dx2, dx3)
    fidx = (fx0, fx1, fx2, fx3)
    ldidx = (lx0, lx1, lx2, lx3)
    rows = (rw0, rw1, rw2, rw3)
    semi = (mi0, mi1, mi2, mi3)
    semg = (mg0, mg1, mg2, mg3)
    sems = (ms0, ms1, ms2, ms3)
    s = lax.axis_index("s")
    c = lax.axis_index("c")
    zero = jnp.zeros((_L,), jnp.float32)
    row0 = s * _RPT
    base = s * _EPT

    def _rel(ysr, es, ed, r):
        def _ldidx(i, b):
            off = pl.multiple_of(base + i * _CHE, 8)
            pltpu.async_copy(es.at[pl.ds(off, _CHE)], sidx[b], semi[b])
            pltpu.async_copy(ed.at[pl.ds(off, _CHE)], didx[b], semi[b])

        def _filter(b, half0):
            for k in range(_CHE // _L):
                sl = pl.ds(k * _L, _L)
                lv = didx[b][sl] - half0
                keep = (lv >= 0) & (lv < _HN)
                ldidx[b][sl] = jnp.where(keep, lv, -1)
                fidx[b][sl] = jnp.where(keep, sidx[b][sl], -1)

        def _gather(ysr, b):
            pltpu.async_copy(
                ysr.at[plsc.Indices(fidx[b], ignored_value=-1)], rows[b],
                semg[b],
            )

        def _scat_desc(b):
            return pltpu.make_async_copy(
                rows[b], atab.at[plsc.Indices(ldidx[b], ignored_value=-1)],
                sems[b],
            )

        for p in range(2):
            half0 = p * _HN

            def _zfill(i, _):
                for k in range(_D // _L):
                    stg[i, pl.ds(k * _L, _L)] = zero
                return 0

            lax.fori_loop(0, _RPT // 2, _zfill, 0)
            for q in range(2):
                pltpu.sync_copy(
                    stg, atab.at[pl.ds(row0 + q * (_RPT // 2), _RPT // 2)])
            plsc.subcore_barrier()

            for b in range(4):
                _ldidx(b, b)

            def _loop(j, _):
                for b in range(4):
                    i = j * 4 + b
                    pltpu.make_async_copy(
                        es.at[pl.ds(0, _CHE)], sidx[b], semi[b]).wait()
                    pltpu.make_async_copy(
                        es.at[pl.ds(0, _CHE)], didx[b], semi[b]).wait()

                    @pl.when(j >= 1)
                    def _():
                        _scat_desc(b).wait()

                    _filter(b, half0)
                    _gather(ysr, b)

                    @pl.when(j < _NCHE // 4 - 1)
                    def _():
                        _ldidx(i + 4, b)

                for b in range(4):
                    pltpu.make_async_copy(
                        ysr.at[plsc.Indices(fidx[b], ignored_value=-1)],
                        rows[b], semg[b]).wait()
                    pltpu.async_copy(
                        rows[b],
                        atab.at[plsc.Indices(ldidx[b], ignored_value=-1)],
                        sems[b], add=True,
                    )

                return 0

            lax.fori_loop(0, _NCHE // 4, _loop, 0)
            for b in range(4):
                _scat_desc(b).wait()
            toff = pl.multiple_of(base + _NCHE * _CHE, 8)
            pltpu.sync_copy(es.at[pl.ds(toff, 32)], sidx[0].at[pl.ds(0, 32)])
            pltpu.sync_copy(ed.at[pl.ds(toff, 32)], didx[0].at[pl.ds(0, 32)])
            sent = jnp.full((_L,), -_NP, jnp.int32)
            for k in range(2, _CHE // _L):
                didx[0][pl.ds(k * _L, _L)] = sent
            _filter(0, half0)
            pltpu.sync_copy(
                ysr.at[plsc.Indices(fidx[0], ignored_value=-1)], rows[0]
            )
            pltpu.sync_copy(
                rows[0], atab.at[plsc.Indices(ldidx[0], ignored_value=-1)],
                add=True,
            )
            plsc.subcore_barrier()
            for q in range(2):
                pltpu.sync_copy(
                    atab.at[pl.ds(row0 + q * (_RPT // 2), _RPT // 2)], stg)
                pltpu.sync_copy(
                    stg,
                    agg_out.at[r, pl.ds(half0 + row0 + q * (_RPT // 2),
                                        _RPT // 2)])

    @pl.when(c == 0)
    def _():
        _rel(ys0, es0, ed0, 0)

    @pl.when(c == 1)
    def _():
        _rel(ys1, es1, ed1, 1)


_edge_kernel = functools.partial(
    pl.kernel,
    out_type=jax.ShapeDtypeStruct((2, _NP, _D), jnp.float32),
    mesh=_mesh,
    scratch_types=[pltpu.VMEM((_RPT // 2, _D), jnp.float32)]
    + [pltpu.VMEM((_CHE,), jnp.int32) for _ in range(16)]
    + [pltpu.VMEM((_CHE, _D), jnp.float32) for _ in range(4)]
    + [pltpu.SemaphoreType.DMA for _ in range(12)]
    + [pltpu.VMEM_SHARED((_HN, _D), jnp.float32)],
)(_edge_body)


def _mm_body(x_ref, w_ref, y_ref):
    y_ref[...] = jax.lax.dot_general(
        x_ref[...], w_ref[...], (((1,), (0,)), ((), ())),
        preferred_element_type=jnp.float32,
        precision=jax.lax.Precision.HIGHEST,
    )


_mm_kernel = pl.pallas_call(
    _mm_body,
    grid=(_N // _BN,),
    in_specs=[
        pl.BlockSpec((_BN, _D), lambda i: (i, 0)),
        pl.BlockSpec((_D, 2 * _D), lambda i: (0, 0)),
    ],
    out_specs=pl.BlockSpec((_BN, 2 * _D), lambda i: (i, 0)),
    out_shape=jax.ShapeDtypeStruct((_N, 2 * _D), jnp.float32),
)


def _scale_body(y_ref, deg_ref, y0_ref, y1_ref):
    y = y_ref[...]
    d0 = deg_ref[:, 0:1]
    ns0 = lax.rsqrt(jnp.where(d0 > 0, d0, 1.0))
    d1 = deg_ref[:, 2:3]
    ns1 = lax.rsqrt(jnp.where(d1 > 0, d1, 1.0))
    y0_ref[...] = y[:, :_D] * ns0
    y1_ref[...] = y[:, _D:] * ns1


_scale_kernel = pl.pallas_call(
    _scale_body,
    grid=(_N // _BN,),
    in_specs=[
        pl.BlockSpec((_BN, 2 * _D), lambda i: (i, 0)),
        pl.BlockSpec((_BN, 4), lambda i: (i, 0)),
    ],
    out_specs=[
        pl.BlockSpec((_BN, _D), lambda i: (i, 0)),
        pl.BlockSpec((_BN, _D), lambda i: (i, 0)),
    ],
    out_shape=[
        jax.ShapeDtypeStruct((_NP, _D), jnp.float32),
        jax.ShapeDtypeStruct((_NP, _D), jnp.float32),
    ],
)


def _final_body(agg_ref, deg_ref, b0_ref, b1_ref, a_ref, out_ref):
    d0 = deg_ref[:, 1:2]
    nd0 = lax.rsqrt(jnp.where(d0 > 0, d0, 1.0))
    d1 = deg_ref[:, 3:4]
    nd1 = lax.rsqrt(jnp.where(d1 > 0, d1, 1.0))
    h = agg_ref[0] * nd0 + agg_ref[1] * nd1 + b0_ref[...] + b1_ref[...]
    out_ref[...] = jnp.where(h > 0, h, a_ref[...] * h)


_final_kernel = pl.pallas_call(
    _final_body,
    grid=(_N // _BN,),
    in_specs=[
        pl.BlockSpec((2, _BN, _D), lambda i: (0, i, 0)),
        pl.BlockSpec((_BN, 4), lambda i: (i, 0)),
        pl.BlockSpec((1, _D), lambda i: (0, 0)),
        pl.BlockSpec((1, _D), lambda i: (0, 0)),
        pl.BlockSpec((1, 1), lambda i: (0, 0)),
    ],
    out_specs=pl.BlockSpec((_BN, _D), lambda i: (i, 0)),
    out_shape=jax.ShapeDtypeStruct((_N, _D), jnp.float32),
)


def kernel(x, edge_index_rel0, edge_index_rel1, W0, b0, W1, b1, prelu_a):
    es0, ed0 = edge_index_rel0[0], edge_index_rel0[1]
    es1, ed1 = edge_index_rel1[0], edge_index_rel1[1]
    w_cat = jnp.concatenate([W0, W1], axis=1)
    y01 = _mm_kernel(x, w_cat)
    deg = _deg_kernel(es0, ed0, es1, ed1).reshape(4, _N)
    deg_t = deg.T
    ys0, ys1 = _scale_kernel(y01, deg_t)
    agg = _edge_kernel(ys0, ys1, es0, ed0, es1, ed1)
    return _final_kernel(
        agg, deg_t, b0.reshape(1, _D), b1.reshape(1, _D), prelu_a.reshape(1, 1)
    )
